# baseline (speedup 1.0000x reference)
import jax
import jax.numpy as jnp
from jax import lax
from jax.experimental import pallas as pl
from jax.experimental.pallas import tpu as pltpu
from jax.experimental.pallas import tpu_sc as plsc

_B, _D = 8192, 2048
_N_IN = 4
_NC, _NS = 2, 16
_NW = _NC * _NS
_ROWS_W = _B // _NW
_NSLOT = 2
_SIZES = [32, 24, 32, 24, 32, 24, 32, 24, 32]
_OFFS = [sum(_SIZES[:i]) for i in range(len(_SIZES))]
_CHUNKS = list(zip(_OFFS, _SIZES))
_NCH = len(_CHUNKS)


def _mux_body(x0, x1, x2, x3, sel_hbm, out, sel_v, b0, b1, *sems):
    xs = (x0, x1, x2, x3)
    bufs = (b0, b1)
    rsems = sems[:_NSLOT]
    wsems = sems[_NSLOT:]

    sid = lax.axis_index("s")
    wid = sid * _NC + lax.axis_index("c")
    base = wid * _ROWS_W

    pltpu.sync_copy(sel_hbm, sel_v)
    selv = sel_v[...]
    preds = [jnp.any(selv == i) for i in range(_N_IN)]

    def rows(c):
        off, n = _CHUNKS[c]
        return pl.ds(base + off, n)

    def buf(k, c):
        return bufs[k]

    def start_read(c):
        k = c % _NSLOT
        for i in range(_N_IN):
            @pl.when(preds[i])
            def _(i=i, k=k, c=c):
                pltpu.async_copy(xs[i].at[rows(c)], buf(k, c), rsems[k])

    def wait_read(c):
        k = c % _NSLOT
        pltpu.make_async_copy(xs[0].at[rows(c)], buf(k, c), rsems[k]).wait()

    def start_write(c):
        k = c % _NSLOT
        pltpu.async_copy(buf(k, c), out.at[rows(c)], wsems[k])

    def wait_write(c):
        k = c % _NSLOT
        pltpu.make_async_copy(buf(k, c), out.at[rows(c)], wsems[k]).wait()

    for c in range(min(_NSLOT, _NCH)):
        start_read(c)

    for c in range(_NCH):
        wait_read(c)
        start_write(c)
        prev = c - 1
        nxt = prev + _NSLOT
        if prev >= 0 and nxt < _NCH:
            wait_write(prev)
            start_read(nxt)
    for c in range(max(0, _NCH - _NSLOT), _NCH):
        wait_write(c)


def kernel(x0, x1, x2, x3, sel):
    sel_arr = jnp.full((16,), sel, dtype=jnp.int32)
    mesh = plsc.VectorSubcoreMesh(
        core_axis_name="c", subcore_axis_name="s",
        num_cores=_NC, num_subcores=_NS)
    mux = pl.kernel(
        _mux_body,
        out_type=jax.ShapeDtypeStruct((_B, _D), jnp.float32),
        mesh=mesh,
        compiler_params=pltpu.CompilerParams(needs_layout_passes=False),
        scratch_types=(
            [pltpu.VMEM((16,), jnp.int32)]
            + [pltpu.VMEM((32, _D), jnp.float32),
               pltpu.VMEM((24, _D), jnp.float32)]
            + [pltpu.SemaphoreType.DMA for _ in range(2 * _NSLOT)]
        ),
    )
    return mux(x0, x1, x2, x3, sel_arr)


# speedup vs baseline: 1.0323x; 1.0323x over previous
import jax
import jax.numpy as jnp
from jax import lax
from jax.experimental import pallas as pl
from jax.experimental.pallas import tpu as pltpu
from jax.experimental.pallas import tpu_sc as plsc

_B, _D = 8192, 2048
_N_IN = 4
_NC, _NS = 2, 16
_NW = _NC * _NS
_ROWS_W = _B // _NW
_CHUNK = 16
_NCH = _ROWS_W // _CHUNK
_NSLOT = 3


def _mux_body(x0, x1, x2, x3, sel_hbm, out, sel_v, stage_sh, *sems):
    xs = (x0, x1, x2, x3)
    rsems = sems[:_NSLOT]
    wsems = sems[_NSLOT:]

    sid = lax.axis_index("s")
    wid = sid * _NC + lax.axis_index("c")
    base = wid * _ROWS_W

    pltpu.sync_copy(sel_hbm, sel_v)
    selv = sel_v[...]
    preds = [jnp.any(selv == i) for i in range(_N_IN)]

    def rows(c):
        return pl.ds(base + c * _CHUNK, _CHUNK)

    def buf(k):
        return stage_sh.at[sid, k]

    def start_read(c):
        k = c % _NSLOT
        for i in range(_N_IN):
            @pl.when(preds[i])
            def _(i=i, k=k, c=c):
                pltpu.async_copy(xs[i].at[rows(c)], buf(k), rsems[k])

    def wait_read(c):
        k = c % _NSLOT
        pltpu.make_async_copy(xs[0].at[rows(c)], buf(k), rsems[k]).wait()

    def start_write(c):
        k = c % _NSLOT
        pltpu.async_copy(buf(k), out.at[rows(c)], wsems[k])

    def wait_write(c):
        k = c % _NSLOT
        pltpu.make_async_copy(buf(k), out.at[rows(c)], wsems[k]).wait()

    for c in range(min(_NSLOT, _NCH)):
        start_read(c)

    for c in range(_NCH):
        wait_read(c)
        start_write(c)
        prev = c - 1
        nxt = prev + _NSLOT
        if prev >= 0 and nxt < _NCH:
            wait_write(prev)
            start_read(nxt)
    for c in range(max(0, _NCH - _NSLOT), _NCH):
        wait_write(c)


def kernel(x0, x1, x2, x3, sel):
    sel_arr = jnp.full((16,), sel, dtype=jnp.int32)
    mesh = plsc.VectorSubcoreMesh(
        core_axis_name="c", subcore_axis_name="s",
        num_cores=_NC, num_subcores=_NS)
    mux = pl.kernel(
        _mux_body,
        out_type=jax.ShapeDtypeStruct((_B, _D), jnp.float32),
        mesh=mesh,
        compiler_params=pltpu.CompilerParams(needs_layout_passes=False),
        scratch_types=(
            [pltpu.VMEM((16,), jnp.int32),
             pltpu.MemorySpace.VMEM_SHARED((_NS, _NSLOT, _CHUNK, _D),
                                           jnp.float32)]
            + [pltpu.SemaphoreType.DMA for _ in range(2 * _NSLOT)]
        ),
    )
    return mux(x0, x1, x2, x3, sel_arr)
